# baseline (speedup 1.0000x reference)
import functools

import jax
import jax.numpy as jnp
from jax import lax
from jax.experimental import pallas as pl
from jax.experimental.pallas import tpu as pltpu
from jax.experimental.pallas import tpu_sc as plsc


def _sc_gather(table, idx_flat):
    n, = idx_flat.shape
    _, d = table.shape

    info = plsc.get_sparse_core_info()
    nw = info.num_cores * info.num_subcores
    nc = info.num_cores

    chunk = 128
    per_w = n // nw
    n_chunks = per_w // chunk
    assert per_w % chunk == 0 and n % nw == 0

    idx3 = idx_flat.reshape(nw, n_chunks, chunk)
    mesh = plsc.VectorSubcoreMesh(core_axis_name="c", subcore_axis_name="s")

    @functools.partial(
        pl.kernel,
        mesh=mesh,
        compiler_params=pltpu.CompilerParams(use_tc_tiling_on_sc=False),
        out_type=jax.ShapeDtypeStruct((n, d), jnp.float32),
        scratch_types=[
            pltpu.VMEM((n_chunks, chunk), jnp.int32),
            pltpu.VMEM((per_w, d), jnp.float32),
            pltpu.SemaphoreType.DMA,
        ],
    )
    def gather_k(table_hbm, idx_hbm, out_hbm, idx_v, rows_v, sem):
        wid = lax.axis_index("s") * nc + lax.axis_index("c")
        base = wid * per_w
        pltpu.sync_copy(idx_hbm.at[wid], idx_v)
        copies = []
        for j in range(n_chunks):
            copies.append(
                pltpu.async_copy(
                    table_hbm.at[idx_v.at[j]],
                    rows_v.at[pl.ds(j * chunk, chunk)],
                    sem,
                )
            )
        for c in copies:
            c.wait()
        pltpu.sync_copy(rows_v, out_hbm.at[pl.ds(base, per_w)])

    return gather_k(table, idx3)


def _mlp_body(e_ref, w1_ref, b1_ref, w2_ref, b2_ref, o_ref, h_ref):
    @pl.when(pl.program_id(0) == 0)
    def _():
        h = jnp.dot(e_ref[...], w1_ref[...], preferred_element_type=jnp.float32)
        h_ref[...] = jnp.maximum(h + b1_ref[...], 0.0)

    o_ref[...] = (
        jnp.dot(h_ref[...], w2_ref[...], preferred_element_type=jnp.float32)
        + b2_ref[...]
    )


def _mlp(e, w1, b1, w2, b2, block_v):
    b, wd = e.shape
    h = w1.shape[1]
    v = w2.shape[1]
    grid = (pl.cdiv(v, block_v),)
    return pl.pallas_call(
        _mlp_body,
        grid=grid,
        in_specs=[
            pl.BlockSpec((b, wd), lambda i: (0, 0)),
            pl.BlockSpec((wd, h), lambda i: (0, 0)),
            pl.BlockSpec((1, h), lambda i: (0, 0)),
            pl.BlockSpec((h, block_v), lambda i: (0, i)),
            pl.BlockSpec((1, block_v), lambda i: (0, i)),
        ],
        out_specs=pl.BlockSpec((b, block_v), lambda i: (0, i)),
        out_shape=jax.ShapeDtypeStruct((b, v), jnp.float32),
        scratch_shapes=[pltpu.VMEM((b, h), jnp.float32)],
    )(e, w1, b1.reshape(1, h), w2, b2.reshape(1, v))


def kernel(x, embed_table, W1, b1, W2, b2):
    bsz, w = x.shape
    _, d = embed_table.shape
    idx = x.reshape(-1).astype(jnp.int32)
    rows = _sc_gather(embed_table, idx)
    e = rows.reshape(bsz, w * d)
    return _mlp(e, W1, b1, W2, b2, block_v=2048)


# speedup vs baseline: 1.0022x; 1.0022x over previous
import functools

import jax
import jax.numpy as jnp
from jax import lax
from jax.experimental import pallas as pl
from jax.experimental.pallas import tpu as pltpu
from jax.experimental.pallas import tpu_sc as plsc


def _sc_gather(table, idx_flat):
    n, = idx_flat.shape
    _, d = table.shape

    info = plsc.get_sparse_core_info()
    nw = info.num_cores * info.num_subcores
    nc = info.num_cores

    chunk = 128
    per_w = n // nw
    n_chunks = per_w // chunk
    assert per_w % chunk == 0 and n % nw == 0

    idx3 = idx_flat.reshape(nw, n_chunks, chunk)
    mesh = plsc.VectorSubcoreMesh(core_axis_name="c", subcore_axis_name="s")

    @functools.partial(
        pl.kernel,
        mesh=mesh,
        compiler_params=pltpu.CompilerParams(use_tc_tiling_on_sc=False),
        out_type=jax.ShapeDtypeStruct((n, d), jnp.float32),
        scratch_types=[
            pltpu.VMEM((n_chunks, chunk), jnp.int32),
            pltpu.VMEM((per_w, d), jnp.float32),
            pltpu.SemaphoreType.DMA,
        ],
    )
    def gather_k(table_hbm, idx_hbm, out_hbm, idx_v, rows_v, sem):
        wid = lax.axis_index("s") * nc + lax.axis_index("c")
        base = wid * per_w
        pltpu.sync_copy(idx_hbm.at[wid], idx_v)
        copies = []
        for j in range(n_chunks):
            copies.append(
                pltpu.async_copy(
                    table_hbm.at[idx_v.at[j]],
                    rows_v.at[pl.ds(j * chunk, chunk)],
                    sem,
                )
            )
        for c in copies:
            c.wait()
        pltpu.sync_copy(rows_v, out_hbm.at[pl.ds(base, per_w)])

    return gather_k(table, idx3)


def _mlp_body(e_ref, w1_ref, b1_ref, w2_ref, b2_ref, o_ref, h_ref):
    @pl.when(pl.program_id(0) == 0)
    def _():
        h = jnp.dot(e_ref[...], w1_ref[...], preferred_element_type=jnp.float32)
        h_ref[...] = jnp.maximum(h + b1_ref[...], 0.0)

    o_ref[...] = (
        jnp.dot(h_ref[...], w2_ref[...], preferred_element_type=jnp.float32)
        + b2_ref[...]
    )


def _mlp(e, w1, b1, w2, b2, block_v):
    b, wd = e.shape
    h = w1.shape[1]
    v = w2.shape[1]
    grid = (pl.cdiv(v, block_v),)
    return pl.pallas_call(
        _mlp_body,
        grid=grid,
        in_specs=[
            pl.BlockSpec((b, wd), lambda i: (0, 0)),
            pl.BlockSpec((wd, h), lambda i: (0, 0)),
            pl.BlockSpec((1, h), lambda i: (0, 0)),
            pl.BlockSpec((h, block_v), lambda i: (0, i)),
            pl.BlockSpec((1, block_v), lambda i: (0, i)),
        ],
        out_specs=pl.BlockSpec((b, block_v), lambda i: (0, i)),
        out_shape=jax.ShapeDtypeStruct((b, v), jnp.float32),
        scratch_shapes=[pltpu.VMEM((b, h), jnp.float32)],
    )(e, w1, b1.reshape(1, h), w2, b2.reshape(1, v))


def kernel(x, embed_table, W1, b1, W2, b2):
    bsz, w = x.shape
    _, d = embed_table.shape
    idx = x.reshape(-1).astype(jnp.int32)
    rows = _sc_gather(embed_table, idx)
    e = rows.reshape(bsz, w * d)
    return _mlp(e, W1, b1, W2, b2, block_v=4096)
